# baseline (speedup 1.0000x reference)
import functools

import jax
import jax.numpy as jnp
from jax import lax
from jax.experimental import pallas as pl
from jax.experimental.pallas import tpu as pltpu
from jax.experimental.pallas import tpu_sc as plsc

_D = 32
_K = 8192
_H = _K // 2
_R = 1024
_COMMIT = 0.25
_BIG = 2 ** 30


def _dist_argmin_body(x_ref, c_ref, a_ref, b_ref, idx_ref, loss_ref):
    i = pl.program_id(0)
    x = x_ref[...]
    c = c_ref[...]
    a = a_ref[...]
    b = b_ref[...]
    m = lax.dot_general(x, c, (((1,), (1,)), ((), ())),
                        preferred_element_type=jnp.float32)
    d = (a[:, None] + b[None, :]) - 2.0 * m
    dA = d[:, :_H]
    dB = d[:, _H:]
    mnA = jnp.min(dA, axis=1, keepdims=True)
    mnB = jnp.min(dB, axis=1, keepdims=True)
    iota = lax.broadcasted_iota(jnp.int32, (_R, _H), 1)
    idxA = jnp.min(jnp.where(dA == mnA, iota, _BIG), axis=1)
    idxB = jnp.min(jnp.where(dB == mnB, iota + _H, _BIG), axis=1)
    mnA_bf = mnA.astype(jnp.bfloat16).astype(jnp.float32)
    steal = mnB < mnA_bf
    idx_ref[...] = jnp.where(steal[:, 0], idxB, idxA)
    val = jnp.where(steal, mnB, mnA)

    @pl.when(i == 0)
    def _():
        loss_ref[0, 0] = 0.0

    loss_ref[0, 0] += jnp.sum(val)


def _tc_dist_argmin(xb, cb, a, b):
    n = xb.shape[0]
    nb = n // _R
    return pl.pallas_call(
        _dist_argmin_body,
        grid=(nb,),
        in_specs=[
            pl.BlockSpec((_R, _D), lambda i: (i, 0)),
            pl.BlockSpec((_K, _D), lambda i: (0, 0)),
            pl.BlockSpec((_R,), lambda i: (i,)),
            pl.BlockSpec((_K,), lambda i: (0,)),
        ],
        out_specs=[
            pl.BlockSpec((_R,), lambda i: (i,)),
            pl.BlockSpec(memory_space=pltpu.SMEM),
        ],
        out_shape=[
            jax.ShapeDtypeStruct((n,), jnp.int32),
            jax.ShapeDtypeStruct((1, 1), jnp.float32),
        ],
    )(xb, cb, a, b)


def _sc_gather(codebook, idx):
    info = plsc.get_sparse_core_info()
    nw = info.num_cores * info.num_subcores
    n = idx.shape[0]
    b_per_w = n // nw
    mesh = plsc.VectorSubcoreMesh(core_axis_name="c", subcore_axis_name="s")

    @functools.partial(
        pl.kernel,
        mesh=mesh,
        out_type=jax.ShapeDtypeStruct((n, _D), jnp.float32),
        scratch_types=[
            pltpu.VMEM((b_per_w,), jnp.int32),
            pltpu.VMEM((b_per_w, _D), jnp.float32),
            pltpu.SemaphoreType.DMA,
        ],
        compiler_params=pltpu.CompilerParams(use_tc_tiling_on_sc=False),
    )
    def gather_kernel(table_hbm, idx_hbm, out_hbm, idx_v, rows_v, sem):
        wid = lax.axis_index("s") * info.num_cores + lax.axis_index("c")
        base = wid * b_per_w
        pltpu.sync_copy(idx_hbm.at[pl.ds(base, b_per_w)], idx_v)
        pltpu.async_copy(table_hbm.at[idx_v], rows_v, sem).wait()
        pltpu.sync_copy(rows_v, out_hbm.at[pl.ds(base, b_per_w)])

    return gather_kernel(codebook, idx)


def kernel(inputs, codebook):
    flat = inputs.reshape(-1, _D)
    n = flat.shape[0]
    a = jnp.sum(flat ** 2, axis=1)
    b = jnp.sum(codebook ** 2, axis=1)
    xb = flat.astype(jnp.bfloat16)
    cb = codebook.astype(jnp.bfloat16)
    idx, loss_sum = _tc_dist_argmin(xb, cb, a, b)
    q = _sc_gather(codebook, idx)
    quantized = q.reshape(inputs.shape)
    vq_loss = (1.0 + _COMMIT) * loss_sum[0, 0] / (n * _D)
    return quantized, vq_loss, idx.reshape(inputs.shape[:-1])


# speedup vs baseline: 1.0095x; 1.0095x over previous
import functools

import jax
import jax.numpy as jnp
from jax import lax
from jax.experimental import pallas as pl
from jax.experimental.pallas import tpu as pltpu
from jax.experimental.pallas import tpu_sc as plsc

_D = 32
_K = 8192
_H = _K // 2
_R = 512
_COMMIT = 0.25
_BIG = 2 ** 30


def _dist_argmin_body(x_ref, c_ref, a_ref, b_ref, idx_ref, loss_ref):
    i = pl.program_id(0)
    x = x_ref[...]
    c = c_ref[...]
    a = a_ref[...]
    b = b_ref[...]
    m = lax.dot_general(x, c, (((1,), (1,)), ((), ())),
                        preferred_element_type=jnp.float32)
    d = (a[:, None] + b[None, :]) - 2.0 * m
    dA = d[:, :_H]
    dB = d[:, _H:]
    mnA = jnp.min(dA, axis=1, keepdims=True)
    mnB = jnp.min(dB, axis=1, keepdims=True)
    iota = lax.broadcasted_iota(jnp.int32, (_R, _H), 1)
    idxA = jnp.min(jnp.where(dA == mnA, iota, _BIG), axis=1)
    idxB = jnp.min(jnp.where(dB == mnB, iota + _H, _BIG), axis=1)
    mnA_bf = mnA.astype(jnp.bfloat16).astype(jnp.float32)
    steal = mnB < mnA_bf
    idx_ref[...] = jnp.where(steal[:, 0], idxB, idxA)
    val = jnp.where(steal, mnB, mnA)

    @pl.when(i == 0)
    def _():
        loss_ref[0, 0] = 0.0

    loss_ref[0, 0] += jnp.sum(val)


def _tc_dist_argmin(xb, cb, a, b):
    n = xb.shape[0]
    nb = n // _R
    return pl.pallas_call(
        _dist_argmin_body,
        grid=(nb,),
        in_specs=[
            pl.BlockSpec((_R, _D), lambda i: (i, 0)),
            pl.BlockSpec((_K, _D), lambda i: (0, 0)),
            pl.BlockSpec((_R,), lambda i: (i,)),
            pl.BlockSpec((_K,), lambda i: (0,)),
        ],
        out_specs=[
            pl.BlockSpec((_R,), lambda i: (i,)),
            pl.BlockSpec(memory_space=pltpu.SMEM),
        ],
        out_shape=[
            jax.ShapeDtypeStruct((n,), jnp.int32),
            jax.ShapeDtypeStruct((1, 1), jnp.float32),
        ],
    )(xb, cb, a, b)


def _sc_gather(codebook, idx):
    info = plsc.get_sparse_core_info()
    nw = info.num_cores * info.num_subcores
    n = idx.shape[0]
    b_per_w = n // nw
    mesh = plsc.VectorSubcoreMesh(core_axis_name="c", subcore_axis_name="s")

    @functools.partial(
        pl.kernel,
        mesh=mesh,
        out_type=jax.ShapeDtypeStruct((n, _D), jnp.float32),
        scratch_types=[
            pltpu.VMEM((b_per_w,), jnp.int32),
            pltpu.VMEM((b_per_w, _D), jnp.float32),
            pltpu.SemaphoreType.DMA,
        ],
        compiler_params=pltpu.CompilerParams(use_tc_tiling_on_sc=False),
    )
    def gather_kernel(table_hbm, idx_hbm, out_hbm, idx_v, rows_v, sem):
        wid = lax.axis_index("s") * info.num_cores + lax.axis_index("c")
        base = wid * b_per_w
        pltpu.sync_copy(idx_hbm.at[pl.ds(base, b_per_w)], idx_v)
        pltpu.async_copy(table_hbm.at[idx_v], rows_v, sem).wait()
        pltpu.sync_copy(rows_v, out_hbm.at[pl.ds(base, b_per_w)])

    return gather_kernel(codebook, idx)


def kernel(inputs, codebook):
    flat = inputs.reshape(-1, _D)
    n = flat.shape[0]
    a = jnp.sum(flat ** 2, axis=1)
    b = jnp.sum(codebook ** 2, axis=1)
    xb = flat.astype(jnp.bfloat16)
    cb = codebook.astype(jnp.bfloat16)
    idx, loss_sum = _tc_dist_argmin(xb, cb, a, b)
    q = _sc_gather(codebook, idx)
    quantized = q.reshape(inputs.shape)
    vq_loss = (1.0 + _COMMIT) * loss_sum[0, 0] / (n * _D)
    return quantized, vq_loss, idx.reshape(inputs.shape[:-1])


# speedup vs baseline: 1.0123x; 1.0028x over previous
import functools

import jax
import jax.numpy as jnp
from jax import lax
from jax.experimental import pallas as pl
from jax.experimental.pallas import tpu as pltpu
from jax.experimental.pallas import tpu_sc as plsc

_D = 32
_K = 8192
_H = _K // 2
_R = 512
_COMMIT = 0.25
_BIG = 2 ** 30


def _dist_argmin_body(x_ref, c_ref, a_ref, b_ref, idx_ref, loss_ref):
    i = pl.program_id(0)
    x = x_ref[...]
    c = c_ref[...]
    a = a_ref[...]
    b = b_ref[...]
    m = lax.dot_general(x, c, (((1,), (1,)), ((), ())),
                        preferred_element_type=jnp.float32)
    d = (a[:, None] + b[None, :]) - 2.0 * m
    dA = d[:, :_H]
    dB = d[:, _H:]
    mnA = jnp.min(dA, axis=1, keepdims=True)
    mnB = jnp.min(dB, axis=1, keepdims=True)
    iota = lax.broadcasted_iota(jnp.int32, (_R, _H), 1)
    idxA = jnp.min(jnp.where(dA == mnA, iota, _BIG), axis=1)
    idxB = jnp.min(jnp.where(dB == mnB, iota + _H, _BIG), axis=1)
    mnA_bf = mnA.astype(jnp.bfloat16).astype(jnp.float32)
    steal = mnB < mnA_bf
    idx_ref[...] = jnp.where(steal[:, 0], idxB, idxA)
    val = jnp.where(steal, mnB, mnA)
    loss_ref[...] = jnp.full((1, 1, 128), jnp.sum(val), jnp.float32)


def _tc_dist_argmin(xb, cb, a, b):
    n = xb.shape[0]
    nb = n // _R
    return pl.pallas_call(
        _dist_argmin_body,
        grid=(nb,),
        in_specs=[
            pl.BlockSpec((_R, _D), lambda i: (i, 0)),
            pl.BlockSpec((_K, _D), lambda i: (0, 0)),
            pl.BlockSpec((_R,), lambda i: (i,)),
            pl.BlockSpec((_K,), lambda i: (0,)),
        ],
        out_specs=[
            pl.BlockSpec((_R,), lambda i: (i,)),
            pl.BlockSpec((1, 1, 128), lambda i: (i, 0, 0)),
        ],
        out_shape=[
            jax.ShapeDtypeStruct((n,), jnp.int32),
            jax.ShapeDtypeStruct((nb, 1, 128), jnp.float32),
        ],
        compiler_params=pltpu.CompilerParams(
            dimension_semantics=("parallel",)),
    )(xb, cb, a, b)


def _sc_gather(codebook, idx):
    info = plsc.get_sparse_core_info()
    nw = info.num_cores * info.num_subcores
    n = idx.shape[0]
    b_per_w = n // nw
    mesh = plsc.VectorSubcoreMesh(core_axis_name="c", subcore_axis_name="s")

    @functools.partial(
        pl.kernel,
        mesh=mesh,
        out_type=jax.ShapeDtypeStruct((n, _D), jnp.float32),
        scratch_types=[
            pltpu.VMEM((b_per_w,), jnp.int32),
            pltpu.VMEM((b_per_w, _D), jnp.float32),
            pltpu.SemaphoreType.DMA,
        ],
        compiler_params=pltpu.CompilerParams(use_tc_tiling_on_sc=False),
    )
    def gather_kernel(table_hbm, idx_hbm, out_hbm, idx_v, rows_v, sem):
        wid = lax.axis_index("s") * info.num_cores + lax.axis_index("c")
        base = wid * b_per_w
        pltpu.sync_copy(idx_hbm.at[pl.ds(base, b_per_w)], idx_v)
        pltpu.async_copy(table_hbm.at[idx_v], rows_v, sem).wait()
        pltpu.sync_copy(rows_v, out_hbm.at[pl.ds(base, b_per_w)])

    return gather_kernel(codebook, idx)


def kernel(inputs, codebook):
    flat = inputs.reshape(-1, _D)
    n = flat.shape[0]
    a = jnp.sum(flat ** 2, axis=1)
    b = jnp.sum(codebook ** 2, axis=1)
    xb = flat.astype(jnp.bfloat16)
    cb = codebook.astype(jnp.bfloat16)
    idx, loss_parts = _tc_dist_argmin(xb, cb, a, b)
    q = _sc_gather(codebook, idx)
    quantized = q.reshape(inputs.shape)
    vq_loss = (1.0 + _COMMIT) * jnp.sum(loss_parts[:, 0, 0]) / (n * _D)
    return quantized, vq_loss, idx.reshape(inputs.shape[:-1])
